# initial kernel scaffold (unmeasured)
import jax
import jax.numpy as jnp
from jax import lax
from jax.experimental import pallas as pl
from jax.experimental.pallas import tpu as pltpu

B, SQ, H, D = 4, 32, 8, 128
SKV_SHARD = 4096
N_SPLIT = 4
CHUNK = SKV_SHARD // N_SPLIT
SCALE = D ** -0.5
ROWS = B * H * SQ


def kernel(Q, K, V):
    def body(q_ref, k_ref, v_ref, out_ref,
             kbuf, vbuf, acc_num, acc_den, recv_num, recv_den,
             sem_k, sem_v, send_n, recv_n, send_d, recv_d):
        x = lax.axis_index("x")
        y = lax.axis_index("y")
        z = lax.axis_index("z")
        start = (2 * y + z) * CHUNK

        copies = []
        for h in range(H):
            copies.append(pltpu.make_async_copy(
                k_ref.at[:, pl.ds(start, CHUNK), h, :], kbuf.at[:, h], sem_k))
            copies.append(pltpu.make_async_copy(
                v_ref.at[:, pl.ds(start, CHUNK), h, :], vbuf.at[:, h], sem_v))
        for c in copies:
            c.start()

        barrier = pltpu.get_barrier_semaphore()
        for nbr in ((1 - x, y, z), (x, 1 - y, z), (x, y, 1 - z)):
            pl.semaphore_signal(barrier, inc=1, device_id=nbr,
                                device_id_type=pl.DeviceIdType.MESH)
        pl.semaphore_wait(barrier, 3)

        for c in copies:
            c.wait()

        for b in range(B):
            for h in range(H):
                i = b * H + h
                qb = q_ref[b, :, h, :] * SCALE
                s = lax.dot_general(
                    qb, kbuf[b, h], (((1,), (1,)), ((), ())),
                    preferred_element_type=jnp.float32)
                p = jnp.exp(s)
                acc_den[pl.ds(i * SQ, SQ), :] = jnp.sum(p, axis=1,
                                                        keepdims=True)
                acc_num[pl.ds(i * SQ, SQ), :] = lax.dot_general(
                    p, vbuf[b, h], (((1,), (0,)), ((), ())),
                    preferred_element_type=jnp.float32)

        nbrs = ((x, y, 1 - z), (x, 1 - y, z), (1 - x, y, z))
        for ph, tgt in enumerate(nbrs):
            rn = pltpu.make_async_remote_copy(
                src_ref=acc_num, dst_ref=recv_num.at[ph],
                send_sem=send_n.at[ph], recv_sem=recv_n.at[ph],
                device_id=tgt, device_id_type=pl.DeviceIdType.MESH)
            rd = pltpu.make_async_remote_copy(
                src_ref=acc_den, dst_ref=recv_den.at[ph],
                send_sem=send_d.at[ph], recv_sem=recv_d.at[ph],
                device_id=tgt, device_id_type=pl.DeviceIdType.MESH)
            rn.start()
            rd.start()
            rn.wait()
            rd.wait()
            acc_num[...] = acc_num[...] + recv_num[ph]
            acc_den[...] = acc_den[...] + recv_den[ph]

        for b in range(B):
            for h in range(H):
                i = b * H + h
                num = acc_num[pl.ds(i * SQ, SQ), :]
                den = acc_den[pl.ds(i * SQ, SQ), :]
                out_ref[b, :, h, :] = num / den

    return pl.pallas_call(
        body,
        out_shape=jax.ShapeDtypeStruct((B, SQ, H, D), jnp.float32),
        in_specs=[
            pl.BlockSpec(memory_space=pltpu.VMEM),
            pl.BlockSpec(memory_space=pltpu.ANY),
            pl.BlockSpec(memory_space=pltpu.ANY),
        ],
        out_specs=pl.BlockSpec(memory_space=pltpu.VMEM),
        scratch_shapes=[
            pltpu.VMEM((B, H, CHUNK, D), jnp.float32),
            pltpu.VMEM((B, H, CHUNK, D), jnp.float32),
            pltpu.VMEM((ROWS, D), jnp.float32),
            pltpu.VMEM((ROWS, 1), jnp.float32),
            pltpu.VMEM((3, ROWS, D), jnp.float32),
            pltpu.VMEM((3, ROWS, 1), jnp.float32),
            pltpu.SemaphoreType.DMA,
            pltpu.SemaphoreType.DMA,
            pltpu.SemaphoreType.DMA((3,)),
            pltpu.SemaphoreType.DMA((3,)),
            pltpu.SemaphoreType.DMA((3,)),
            pltpu.SemaphoreType.DMA((3,)),
        ],
        compiler_params=pltpu.CompilerParams(collective_id=0),
    )(Q, K, V)


# baseline (device time: 64809 ns/iter reference)
import jax
import jax.numpy as jnp
from jax import lax
from jax.experimental import pallas as pl
from jax.experimental.pallas import tpu as pltpu

B, SQ, H, D = 4, 32, 8, 128
SKV_SHARD = 4096
N_SPLIT = 4
CHUNK = SKV_SHARD // N_SPLIT
SCALE = D ** -0.5
ROWS = B * H * SQ


def kernel(Q, K, V):
    def body(q_ref, k_ref, v_ref, out_ref,
             kbuf, vbuf, acc_num, acc_den, recv_num, recv_den,
             sem_k, sem_v, send_n, recv_n, send_d, recv_d):
        x = lax.axis_index("x")
        y = lax.axis_index("y")
        z = lax.axis_index("z")
        start = (2 * y + z) * CHUNK

        copies = []
        for h in range(H):
            copies.append(pltpu.make_async_copy(
                k_ref.at[:, pl.ds(start, CHUNK), h, :], kbuf.at[:, h], sem_k))
            copies.append(pltpu.make_async_copy(
                v_ref.at[:, pl.ds(start, CHUNK), h, :], vbuf.at[:, h], sem_v))
        for c in copies:
            c.start()

        barrier = pltpu.get_barrier_semaphore()
        for nbr in ((1 - x, y, z), (x, 1 - y, z), (x, y, 1 - z)):
            pl.semaphore_signal(barrier, inc=1, device_id=nbr,
                                device_id_type=pl.DeviceIdType.MESH)
        pl.semaphore_wait(barrier, 3)

        for c in copies:
            c.wait()

        for b in range(B):
            for h in range(H):
                i = b * H + h
                qb = q_ref[b, :, h, :] * SCALE
                s = lax.dot_general(
                    qb, kbuf[b, h], (((1,), (1,)), ((), ())),
                    preferred_element_type=jnp.float32)
                p = jnp.exp(s)
                acc_den[pl.ds(i * SQ, SQ), :] = jnp.sum(p, axis=1,
                                                        keepdims=True)
                acc_num[pl.ds(i * SQ, SQ), :] = lax.dot_general(
                    p, vbuf[b, h], (((1,), (0,)), ((), ())),
                    preferred_element_type=jnp.float32)

        nbrs = ((x, y, 1 - z), (x, 1 - y, z), (1 - x, y, z))
        for ph, tgt in enumerate(nbrs):
            rn = pltpu.make_async_remote_copy(
                src_ref=acc_num, dst_ref=recv_num.at[ph],
                send_sem=send_n.at[ph], recv_sem=recv_n.at[ph],
                device_id=tgt, device_id_type=pl.DeviceIdType.MESH)
            rd = pltpu.make_async_remote_copy(
                src_ref=acc_den, dst_ref=recv_den.at[ph],
                send_sem=send_d.at[ph], recv_sem=recv_d.at[ph],
                device_id=tgt, device_id_type=pl.DeviceIdType.MESH)
            rn.start()
            rd.start()
            rn.wait()
            rd.wait()
            acc_num[...] = acc_num[...] + recv_num[ph]
            acc_den[...] = acc_den[...] + recv_den[ph]

        for b in range(B):
            for h in range(H):
                i = b * H + h
                num = acc_num[pl.ds(i * SQ, SQ), :]
                den = acc_den[pl.ds(i * SQ, SQ), :]
                out_ref[b, :, h, :] = num / den

    return pl.pallas_call(
        body,
        out_shape=jax.ShapeDtypeStruct((B, SQ, H, D), jnp.float32),
        in_specs=[
            pl.BlockSpec(memory_space=pltpu.VMEM),
            pl.BlockSpec(memory_space=pl.ANY),
            pl.BlockSpec(memory_space=pl.ANY),
        ],
        out_specs=pl.BlockSpec(memory_space=pltpu.VMEM),
        scratch_shapes=[
            pltpu.VMEM((B, H, CHUNK, D), jnp.float32),
            pltpu.VMEM((B, H, CHUNK, D), jnp.float32),
            pltpu.VMEM((ROWS, D), jnp.float32),
            pltpu.VMEM((ROWS, 1), jnp.float32),
            pltpu.VMEM((3, ROWS, D), jnp.float32),
            pltpu.VMEM((3, ROWS, 1), jnp.float32),
            pltpu.SemaphoreType.DMA,
            pltpu.SemaphoreType.DMA,
            pltpu.SemaphoreType.DMA((3,)),
            pltpu.SemaphoreType.DMA((3,)),
            pltpu.SemaphoreType.DMA((3,)),
            pltpu.SemaphoreType.DMA((3,)),
        ],
        compiler_params=pltpu.CompilerParams(
            collective_id=0,
            vmem_limit_bytes=100 * 1024 * 1024,
        ),
    )(Q, K, V)


# device time: 36933 ns/iter; 1.7548x vs baseline; 1.7548x over previous
import jax
import jax.numpy as jnp
from jax import lax
from jax.experimental import pallas as pl
from jax.experimental.pallas import tpu as pltpu

B, SQ, H, D = 4, 32, 8, 128
SKV_SHARD = 4096
N_SPLIT = 4
CHUNK = SKV_SHARD // N_SPLIT
SCALE = D ** -0.5
ROWS = B * H * SQ
BH = B * H
TOT = ROWS + BH
THIRD = TOT // 3


def kernel(Q, K, V):
    def body(q_ref, k_ref, v_ref, out_ref,
             kbuf, vbuf, recv, sem_k, sem_v, send_s, recv_s):
        x = lax.axis_index("x")
        y = lax.axis_index("y")
        z = lax.axis_index("z")
        start = (2 * y + z) * CHUNK

        copies = []
        for h in range(H):
            copies.append(pltpu.make_async_copy(
                k_ref.at[:, pl.ds(start, CHUNK), h, :], kbuf.at[:, h], sem_k))
            copies.append(pltpu.make_async_copy(
                v_ref.at[:, pl.ds(start, CHUNK), h, :], vbuf.at[:, h], sem_v))
        for c in copies:
            c.start()

        barrier = pltpu.get_barrier_semaphore()
        for nbr in ((1 - x, y, z), (x, 1 - y, z), (x, y, 1 - z)):
            pl.semaphore_signal(barrier, inc=1, device_id=nbr,
                                device_id_type=pl.DeviceIdType.MESH)
        pl.semaphore_wait(barrier, 3)

        for c in copies:
            c.wait()

        ones_row = jnp.ones((1, CHUNK), jnp.float32)
        for b in range(B):
            for h in range(H):
                i = b * H + h
                qb = q_ref[b, :, h, :] * SCALE
                s = lax.dot_general(
                    qb, kbuf[b, h], (((1,), (1,)), ((), ())),
                    preferred_element_type=jnp.float32)
                p = jnp.exp(s)
                out_ref[pl.ds(i * SQ, SQ), :] = lax.dot_general(
                    p, vbuf[b, h], (((1,), (0,)), ((), ())),
                    preferred_element_type=jnp.float32)
                den_row = lax.dot_general(
                    ones_row, p, (((1,), (1,)), ((), ())),
                    preferred_element_type=jnp.float32)
                out_ref[pl.ds(ROWS + i, 1), :] = jnp.pad(
                    den_row, ((0, 0), (0, D - SQ)))

        nbrs = ((x, y, 1 - z), (x, 1 - y, z), (1 - x, y, z))
        for p_ in range(3):
            rdmas = []
            for j in range(3):
                a = (j + p_) % 3
                r = pltpu.make_async_remote_copy(
                    src_ref=out_ref.at[pl.ds(j * THIRD, THIRD)],
                    dst_ref=recv.at[p_, j],
                    send_sem=send_s.at[p_, j], recv_sem=recv_s.at[p_, j],
                    device_id=nbrs[a], device_id_type=pl.DeviceIdType.MESH)
                r.start()
                rdmas.append(r)
            for r in rdmas:
                r.wait()
            for j in range(3):
                sl = pl.ds(j * THIRD, THIRD)
                out_ref[sl, :] = out_ref[sl, :] + recv[p_, j]

    acc = pl.pallas_call(
        body,
        out_shape=jax.ShapeDtypeStruct((TOT, D), jnp.float32),
        in_specs=[
            pl.BlockSpec(memory_space=pltpu.VMEM),
            pl.BlockSpec(memory_space=pl.ANY),
            pl.BlockSpec(memory_space=pl.ANY),
        ],
        out_specs=pl.BlockSpec(memory_space=pltpu.VMEM),
        scratch_shapes=[
            pltpu.VMEM((B, H, CHUNK, D), jnp.float32),
            pltpu.VMEM((B, H, CHUNK, D), jnp.float32),
            pltpu.VMEM((3, 3, THIRD, D), jnp.float32),
            pltpu.SemaphoreType.DMA,
            pltpu.SemaphoreType.DMA,
            pltpu.SemaphoreType.DMA((3, 3)),
            pltpu.SemaphoreType.DMA((3, 3)),
        ],
        compiler_params=pltpu.CompilerParams(
            collective_id=0,
            vmem_limit_bytes=100 * 1024 * 1024,
        ),
    )(Q, K, V)

    num = acc[:ROWS].reshape(B, H, SQ, D)
    den = acc[ROWS:, :SQ].reshape(B, H, SQ)
    out = num / den[..., None]
    return out.transpose(0, 2, 1, 3)


# device time: 31997 ns/iter; 2.0255x vs baseline; 1.1543x over previous
import jax
import jax.numpy as jnp
from jax import lax
from jax.experimental import pallas as pl
from jax.experimental.pallas import tpu as pltpu

B, SQ, H, D = 4, 32, 8, 128
SKV_SHARD = 4096
N_SPLIT = 4
CHUNK = SKV_SHARD // N_SPLIT
SCALE = D ** -0.5
ROWS = B * H * SQ
BH = B * H
TOT = ROWS + BH
THIRD = TOT // 3


def kernel(Q, K, V):
    def body(q_ref, k_ref, v_ref, out_ref,
             kbuf, vbuf, recv, sem_k, sem_v, send_s, recv_s):
        x = lax.axis_index("x")
        y = lax.axis_index("y")
        z = lax.axis_index("z")
        start = (2 * y + z) * CHUNK

        copies = []
        for h in range(H):
            copies.append((
                pltpu.make_async_copy(
                    k_ref.at[:, pl.ds(start, CHUNK), h, :], kbuf.at[:, h],
                    sem_k.at[h]),
                pltpu.make_async_copy(
                    v_ref.at[:, pl.ds(start, CHUNK), h, :], vbuf.at[:, h],
                    sem_v.at[h]),
            ))
        for ck, cv in copies:
            ck.start()
            cv.start()

        barrier = pltpu.get_barrier_semaphore()
        for nbr in ((1 - x, y, z), (x, 1 - y, z), (x, y, 1 - z)):
            pl.semaphore_signal(barrier, inc=1, device_id=nbr,
                                device_id_type=pl.DeviceIdType.MESH)
        pl.semaphore_wait(barrier, 3)

        ones_row = jnp.ones((1, CHUNK), jnp.float32)
        for h in range(H):
            ck, cv = copies[h]
            ck.wait()
            cv.wait()
            for b in range(B):
                i = b * H + h
                qb = q_ref[b, :, h, :] * SCALE
                s = lax.dot_general(
                    qb, kbuf[b, h], (((1,), (1,)), ((), ())),
                    preferred_element_type=jnp.float32)
                p = jnp.exp(s)
                out_ref[pl.ds(i * SQ, SQ), :] = lax.dot_general(
                    p, vbuf[b, h], (((1,), (0,)), ((), ())),
                    preferred_element_type=jnp.float32)
                den_row = lax.dot_general(
                    ones_row, p, (((1,), (1,)), ((), ())),
                    preferred_element_type=jnp.float32)
                out_ref[pl.ds(ROWS + i, 1), :] = jnp.pad(
                    den_row, ((0, 0), (0, D - SQ)))

        nbrs = ((x, y, 1 - z), (x, 1 - y, z), (1 - x, y, z))

        def rdma(p_, j):
            a = (j + p_) % 3
            return pltpu.make_async_remote_copy(
                src_ref=out_ref.at[pl.ds(j * THIRD, THIRD)],
                dst_ref=recv.at[p_, j],
                send_sem=send_s.at[p_, j], recv_sem=recv_s.at[p_, j],
                device_id=nbrs[a], device_id_type=pl.DeviceIdType.MESH)

        live = [rdma(0, j) for j in range(3)]
        for r in live:
            r.start()
        for p_ in range(3):
            for j in range(3):
                live[j].wait()
                sl = pl.ds(j * THIRD, THIRD)
                out_ref[sl, :] = out_ref[sl, :] + recv[p_, j]
                if p_ < 2:
                    live[j] = rdma(p_ + 1, j)
                    live[j].start()

    acc = pl.pallas_call(
        body,
        out_shape=jax.ShapeDtypeStruct((TOT, D), jnp.float32),
        in_specs=[
            pl.BlockSpec(memory_space=pltpu.VMEM),
            pl.BlockSpec(memory_space=pl.ANY),
            pl.BlockSpec(memory_space=pl.ANY),
        ],
        out_specs=pl.BlockSpec(memory_space=pltpu.VMEM),
        scratch_shapes=[
            pltpu.VMEM((B, H, CHUNK, D), jnp.float32),
            pltpu.VMEM((B, H, CHUNK, D), jnp.float32),
            pltpu.VMEM((3, 3, THIRD, D), jnp.float32),
            pltpu.SemaphoreType.DMA((H,)),
            pltpu.SemaphoreType.DMA((H,)),
            pltpu.SemaphoreType.DMA((3, 3)),
            pltpu.SemaphoreType.DMA((3, 3)),
        ],
        compiler_params=pltpu.CompilerParams(
            collective_id=0,
            vmem_limit_bytes=100 * 1024 * 1024,
        ),
    )(Q, K, V)

    num = acc[:ROWS].reshape(B, H, SQ, D)
    den = acc[ROWS:, :SQ].reshape(B, H, SQ)
    out = num / den[..., None]
    return out.transpose(0, 2, 1, 3)
